# baseline (speedup 1.0000x reference)
import jax
import jax.numpy as jnp
from jax import lax
from jax.experimental import pallas as pl
from jax.experimental.pallas import tpu as pltpu
from jax.experimental.pallas import tpu_sc as plsc

_V = 1000000
_D = 32
_BATCH = 16384
_FIELDS = 26

_INFO = plsc.get_sparse_core_info()
_NC = _INFO.num_cores
_NS = _INFO.num_subcores
_NW = _NC * _NS

_G_FULL = _V // 128
_T4_ROWS = _V // 4

_BW = _BATCH // _NW
_CH = 256
_PITCH = _CH + 2


def _repack_body(wt_hbm, tail_hbm, t4_hbm, *scratch):
    in_v = scratch[0:2]
    fin_v = scratch[2:5]
    gsem = scratch[5:7]
    wsem = scratch[7:10]

    wid = lax.axis_index("s") * _NC + lax.axis_index("c")
    lanes = lax.iota(jnp.int32, 16)

    g0 = wid * 244

    def start_in(chunk, p):
        pltpu.async_copy(
            wt_hbm.at[:, pl.ds((g0 + chunk * 8) * 128, 1024)], in_v[p], gsem[p]
        )

    def wait_in(chunk, p):
        pltpu.make_async_copy(
            wt_hbm.at[:, pl.ds((g0 + chunk * 8) * 128, 1024)], in_v[p], gsem[p]
        ).wait()

    def start_out(chunk, hi, fp):
        row0 = (g0 + chunk * 8 + hi * 4) * 32
        pltpu.async_copy(fin_v[fp], t4_hbm.at[pl.ds(row0, 128), :], wsem[fp])

    def wait_out(fp):
        pltpu.make_async_copy(
            fin_v[fp], t4_hbm.at[pl.ds(0, 128), :], wsem[fp]
        ).wait()

    crow4 = [lax.shift_right_logical(lanes + cb * 16, 2) for cb in range(8)]
    cmod32 = [lax.mul(lax.bitwise_and(lanes + cb * 16, 3), 32) for cb in range(8)]

    def do_group(p, fp, gi_in, gi_fin, k):
        dperm = lax.bitwise_and(lanes + k, 15)
        for d0 in (0, 16):
            drow = dperm + d0
            for cb in range(8):
                ccol = lanes + (gi_in * 128 + cb * 16)
                v = plsc.load_gather(in_v[p], [drow, ccol])
                row = crow4[cb] + gi_fin * 32
                col = cmod32[cb] + drow
                plsc.store_scatter(fin_v[fp], [row, col], v)

    def do_half(p, fp, hi):
        def per_diag(k, _):
            def per_gi(gi4, _):
                do_group(p, fp, hi * 4 + gi4, gi4, k)
                return ()

            lax.fori_loop(0, 4, per_gi, ())
            return ()

        lax.fori_loop(0, 16, per_diag, ())

    start_in(0, 0)

    def block(bi, _):
        base = bi * 6
        for u in range(6):
            c = base + u
            @pl.when(c + 1 < 30)
            def _():
                start_in(c + 1, (u + 1) % 2)
            wait_in(c, u % 2)
            for hi in range(2):
                fp = (2 * u + hi) % 3
                @pl.when(2 * c + hi >= 3)
                def _():
                    wait_out(fp)
                do_half(u % 2, fp, hi)
                start_out(c, hi, fp)
        return ()

    lax.fori_loop(0, 5, block, ())
    for fp in range(3):
        wait_out(fp)

    left = g0 + 240
    pltpu.sync_copy(wt_hbm.at[:, pl.ds(left * 128, 512)], in_v[0].at[:, pl.ds(0, 512)])
    do_half(0, 0, 0)
    pltpu.sync_copy(fin_v[0], t4_hbm.at[pl.ds(left * 32, 128), :])

    @pl.when(wid == 0)
    def _():
        pltpu.sync_copy(
            wt_hbm.at[:, pl.ds(7808 * 128, 512)], in_v[0].at[:, pl.ds(0, 512)]
        )
        do_half(0, 0, 0)
        pltpu.sync_copy(fin_v[0], t4_hbm.at[pl.ds(7808 * 32, 128), :])

    @pl.when(wid == _NW - 1)
    def _():
        pltpu.sync_copy(tail_hbm, in_v[0].at[:, pl.ds(0, 128)])

        def t_diag(k, _):
            do_group(0, 0, 0, 0, k)
            return ()

        lax.fori_loop(0, 16, t_diag, ())
        pltpu.sync_copy(
            fin_v[0].at[pl.ds(16, 16), :], t4_hbm.at[pl.ds(_G_FULL * 32, 16), :]
        )


def _gather_body(idx_hbm, t4_hbm, out_hbm, *scratch):
    idx_v = scratch[0:2]
    idx2_v = scratch[2:4]
    rows_v = scratch[4:6]
    outT_v = scratch[6:8]
    gsem = scratch[8:10]
    wsem = scratch[10:12]

    wid = lax.axis_index("s") * _NC + lax.axis_index("c")
    b0 = wid * _BW
    lanes = lax.iota(jnp.int32, 16)

    def chunk_off(f, c):
        return f * _BATCH + b0 + c * _CH

    def start_fetch(f, c, p):
        off = chunk_off(f, c)
        pltpu.sync_copy(idx_hbm.at[pl.ds(off, _CH)], idx_v[p])

        def shift(i, _):
            idx2_v[p][pl.ds(i * 16, 16)] = lax.shift_right_logical(
                idx_v[p][pl.ds(i * 16, 16)], 2
            )
            return ()

        lax.fori_loop(0, _CH // 16, shift, ())
        pltpu.async_copy(t4_hbm.at[idx2_v[p]], rows_v[p], gsem[p])

    def wait_fetch(p):
        pltpu.make_async_copy(t4_hbm.at[idx2_v[p]], rows_v[p], gsem[p]).wait()

    def extract(p):
        def per_j16(jj, _):
            j0 = jj * 16
            jrow = lanes + j0
            sub32 = lax.mul(lax.bitwise_and(idx_v[p][pl.ds(j0, 16)], 3), 32)

            def per_diag(k, _):
                dperm = lax.bitwise_and(lanes + k, 15)
                for d0 in (0, 16):
                    drow = dperm + d0
                    v = plsc.load_gather(rows_v[p], [jrow, sub32 + drow])
                    plsc.store_scatter(outT_v[p], [drow, jrow], v)
                return ()

            lax.fori_loop(0, 16, per_diag, ())
            return ()

        lax.fori_loop(0, _CH // 16, per_j16, ())

    def start_write(f, c, p):
        pltpu.async_copy(
            outT_v[p].at[:, pl.ds(0, _CH)],
            out_hbm.at[f, :, pl.ds(b0 + c * _CH, _CH)],
            wsem[p],
        )

    def wait_write(p):
        pltpu.make_async_copy(
            outT_v[p].at[:, pl.ds(0, _CH)],
            out_hbm.at[0, :, pl.ds(b0, _CH)],
            wsem[p],
        ).wait()

    start_fetch(0, 0, 0)
    start_fetch(0, 1, 1)

    def per_field(f, _):
        for p in range(2):
            wait_fetch(p)
            @pl.when(f > 0)
            def _():
                wait_write(p)
            extract(p)
            start_write(f, p, p)
            @pl.when(f + 1 < _FIELDS)
            def _():
                start_fetch(f + 1, p, p)
        return ()

    lax.fori_loop(0, _FIELDS, per_field, ())
    wait_write(0)
    wait_write(1)


_MESH = plsc.VectorSubcoreMesh(core_axis_name="c", subcore_axis_name="s")
_PARAMS = pltpu.CompilerParams(use_tc_tiling_on_sc=True, needs_layout_passes=False)


@jax.jit
def kernel(input_, weight):
    wt_t = jnp.transpose(weight)
    tail_t = jnp.transpose(lax.slice(weight, (_V - 128, 0), (_V, _D)))
    idx_flat = jnp.transpose(input_).reshape(_FIELDS * _BATCH)

    t4 = pl.kernel(
        _repack_body,
        out_type=jax.ShapeDtypeStruct((_T4_ROWS, 128), jnp.float32),
        mesh=_MESH,
        scratch_types=(
            [pltpu.VMEM((_D, 1024), jnp.float32) for _ in range(2)]
            + [pltpu.VMEM((128, 128), jnp.float32) for _ in range(3)]
            + [pltpu.SemaphoreType.DMA for _ in range(5)]
        ),
        compiler_params=_PARAMS,
    )(wt_t, tail_t)

    out_t = pl.kernel(
        _gather_body,
        out_type=jax.ShapeDtypeStruct((_FIELDS, _D, _BATCH), jnp.float32),
        mesh=_MESH,
        scratch_types=(
            [pltpu.VMEM((_CH,), jnp.int32) for _ in range(2)]
            + [pltpu.VMEM((_CH,), jnp.int32) for _ in range(2)]
            + [pltpu.VMEM((_CH, 128), jnp.float32) for _ in range(2)]
            + [pltpu.VMEM((_D, _PITCH), jnp.float32) for _ in range(2)]
            + [pltpu.SemaphoreType.DMA for _ in range(4)]
        ),
        compiler_params=_PARAMS,
    )(idx_flat, t4)
    return jnp.transpose(out_t, (2, 0, 1))


# speedup vs baseline: 1.0125x; 1.0125x over previous
import jax
import jax.numpy as jnp
from jax import lax
from jax.experimental import pallas as pl
from jax.experimental.pallas import tpu as pltpu
from jax.experimental.pallas import tpu_sc as plsc

_V = 1000000
_D = 32
_BATCH = 16384
_FIELDS = 26

_INFO = plsc.get_sparse_core_info()
_NC = _INFO.num_cores
_NS = _INFO.num_subcores
_NW = _NC * _NS

_G_FULL = _V // 128
_T4_ROWS = _V // 4

_BW = _BATCH // _NW
_CH = 256
_PITCH = _CH + 2


def _repack_body(wt_hbm, tail_hbm, t4_hbm, *scratch):
    in_v = scratch[0:2]
    fin_v = scratch[2:5]
    gsem = scratch[5:7]
    wsem = scratch[7:10]

    wid = lax.axis_index("s") * _NC + lax.axis_index("c")
    lanes = lax.iota(jnp.int32, 16)

    g0 = wid * 244

    def start_in(chunk, p):
        pltpu.async_copy(
            wt_hbm.at[:, pl.ds((g0 + chunk * 8) * 128, 1024)], in_v[p], gsem[p]
        )

    def wait_in(chunk, p):
        pltpu.make_async_copy(
            wt_hbm.at[:, pl.ds((g0 + chunk * 8) * 128, 1024)], in_v[p], gsem[p]
        ).wait()

    def start_out(chunk, hi, fp):
        row0 = (g0 + chunk * 8 + hi * 4) * 32
        pltpu.async_copy(fin_v[fp], t4_hbm.at[pl.ds(row0, 128), :], wsem[fp])

    def wait_out(fp):
        pltpu.make_async_copy(
            fin_v[fp], t4_hbm.at[pl.ds(0, 128), :], wsem[fp]
        ).wait()

    crow4 = [lax.shift_right_logical(lanes + cb * 16, 2) for cb in range(8)]
    cmod32 = [lax.mul(lax.bitwise_and(lanes + cb * 16, 3), 32) for cb in range(8)]

    def do_group(p, fp, gi_in, gi_fin, k):
        dperm = lax.bitwise_and(lanes + k, 15)
        for d0 in (0, 16):
            drow = dperm + d0
            for cb in range(8):
                ccol = lanes + (gi_in * 128 + cb * 16)
                v = plsc.load_gather(in_v[p], [drow, ccol])
                row = crow4[cb] + gi_fin * 32
                col = cmod32[cb] + drow
                plsc.store_scatter(fin_v[fp], [row, col], v)

    def do_half(p, fp, hi):
        def per_diag(k, _):
            def per_gi(g2, _):
                do_group(p, fp, hi * 4 + g2 * 2, g2 * 2, k)
                do_group(p, fp, hi * 4 + g2 * 2 + 1, g2 * 2 + 1, k)
                return ()

            lax.fori_loop(0, 2, per_gi, ())
            return ()

        lax.fori_loop(0, 16, per_diag, ())

    start_in(0, 0)

    def block(bi, _):
        base = bi * 6
        for u in range(6):
            c = base + u
            @pl.when(c + 1 < 30)
            def _():
                start_in(c + 1, (u + 1) % 2)
            wait_in(c, u % 2)
            for hi in range(2):
                fp = (2 * u + hi) % 3
                @pl.when(2 * c + hi >= 3)
                def _():
                    wait_out(fp)
                do_half(u % 2, fp, hi)
                start_out(c, hi, fp)
        return ()

    lax.fori_loop(0, 5, block, ())
    for fp in range(3):
        wait_out(fp)

    left = g0 + 240
    pltpu.sync_copy(wt_hbm.at[:, pl.ds(left * 128, 512)], in_v[0].at[:, pl.ds(0, 512)])
    do_half(0, 0, 0)
    pltpu.sync_copy(fin_v[0], t4_hbm.at[pl.ds(left * 32, 128), :])

    @pl.when(wid == 0)
    def _():
        pltpu.sync_copy(
            wt_hbm.at[:, pl.ds(7808 * 128, 512)], in_v[0].at[:, pl.ds(0, 512)]
        )
        do_half(0, 0, 0)
        pltpu.sync_copy(fin_v[0], t4_hbm.at[pl.ds(7808 * 32, 128), :])

    @pl.when(wid == _NW - 1)
    def _():
        pltpu.sync_copy(tail_hbm, in_v[0].at[:, pl.ds(0, 128)])

        def t_diag(k, _):
            do_group(0, 0, 0, 0, k)
            return ()

        lax.fori_loop(0, 16, t_diag, ())
        pltpu.sync_copy(
            fin_v[0].at[pl.ds(16, 16), :], t4_hbm.at[pl.ds(_G_FULL * 32, 16), :]
        )


def _gather_body(idx_hbm, t4_hbm, out_hbm, *scratch):
    idx_v = scratch[0:2]
    idx2_v = scratch[2:4]
    rows_v = scratch[4:6]
    outT_v = scratch[6:8]
    gsem = scratch[8:10]
    wsem = scratch[10:12]

    wid = lax.axis_index("s") * _NC + lax.axis_index("c")
    b0 = wid * _BW
    lanes = lax.iota(jnp.int32, 16)

    def chunk_off(f, c):
        return f * _BATCH + b0 + c * _CH

    def start_fetch(f, c, p):
        off = chunk_off(f, c)
        pltpu.sync_copy(idx_hbm.at[pl.ds(off, _CH)], idx_v[p])

        def shift(i, _):
            idx2_v[p][pl.ds(i * 16, 16)] = lax.shift_right_logical(
                idx_v[p][pl.ds(i * 16, 16)], 2
            )
            return ()

        lax.fori_loop(0, _CH // 16, shift, ())
        pltpu.async_copy(t4_hbm.at[idx2_v[p]], rows_v[p], gsem[p])

    def wait_fetch(p):
        pltpu.make_async_copy(t4_hbm.at[idx2_v[p]], rows_v[p], gsem[p]).wait()

    def extract(p):
        def per_j16(jj, _):
            j0 = jj * 16
            jrow = lanes + j0
            sub32 = lax.mul(lax.bitwise_and(idx_v[p][pl.ds(j0, 16)], 3), 32)

            def per_diag(k, _):
                dperm = lax.bitwise_and(lanes + k, 15)
                for d0 in (0, 16):
                    drow = dperm + d0
                    v = plsc.load_gather(rows_v[p], [jrow, sub32 + drow])
                    plsc.store_scatter(outT_v[p], [drow, jrow], v)
                return ()

            lax.fori_loop(0, 16, per_diag, ())
            return ()

        lax.fori_loop(0, _CH // 16, per_j16, ())

    def start_write(f, c, p):
        pltpu.async_copy(
            outT_v[p].at[:, pl.ds(0, _CH)],
            out_hbm.at[f, :, pl.ds(b0 + c * _CH, _CH)],
            wsem[p],
        )

    def wait_write(p):
        pltpu.make_async_copy(
            outT_v[p].at[:, pl.ds(0, _CH)],
            out_hbm.at[0, :, pl.ds(b0, _CH)],
            wsem[p],
        ).wait()

    start_fetch(0, 0, 0)
    start_fetch(0, 1, 1)

    def per_field(f, _):
        for p in range(2):
            wait_fetch(p)
            @pl.when(f > 0)
            def _():
                wait_write(p)
            extract(p)
            start_write(f, p, p)
            @pl.when(f + 1 < _FIELDS)
            def _():
                start_fetch(f + 1, p, p)
        return ()

    lax.fori_loop(0, _FIELDS, per_field, ())
    wait_write(0)
    wait_write(1)


_MESH = plsc.VectorSubcoreMesh(core_axis_name="c", subcore_axis_name="s")
_PARAMS = pltpu.CompilerParams(use_tc_tiling_on_sc=True, needs_layout_passes=False)


@jax.jit
def kernel(input_, weight):
    wt_t = jnp.transpose(weight)
    tail_t = jnp.transpose(lax.slice(weight, (_V - 128, 0), (_V, _D)))
    idx_flat = jnp.transpose(input_).reshape(_FIELDS * _BATCH)

    t4 = pl.kernel(
        _repack_body,
        out_type=jax.ShapeDtypeStruct((_T4_ROWS, 128), jnp.float32),
        mesh=_MESH,
        scratch_types=(
            [pltpu.VMEM((_D, 1024), jnp.float32) for _ in range(2)]
            + [pltpu.VMEM((128, 128), jnp.float32) for _ in range(3)]
            + [pltpu.SemaphoreType.DMA for _ in range(5)]
        ),
        compiler_params=_PARAMS,
    )(wt_t, tail_t)

    out_t = pl.kernel(
        _gather_body,
        out_type=jax.ShapeDtypeStruct((_FIELDS, _D, _BATCH), jnp.float32),
        mesh=_MESH,
        scratch_types=(
            [pltpu.VMEM((_CH,), jnp.int32) for _ in range(2)]
            + [pltpu.VMEM((_CH,), jnp.int32) for _ in range(2)]
            + [pltpu.VMEM((_CH, 128), jnp.float32) for _ in range(2)]
            + [pltpu.VMEM((_D, _PITCH), jnp.float32) for _ in range(2)]
            + [pltpu.SemaphoreType.DMA for _ in range(4)]
        ),
        compiler_params=_PARAMS,
    )(idx_flat, t4)
    return jnp.transpose(out_t, (2, 0, 1))
